# baseline (speedup 1.0000x reference)
import functools
import jax
import jax.numpy as jnp
from jax import lax
from jax.experimental import pallas as pl
from jax.experimental.pallas import tpu as pltpu
from jax.experimental.pallas import tpu_sc as plsc

_BATCH = 128
_VOCAB = 200
_DIM = 128
_NC = 2
_NS = 16
_NW = _NC * _NS
_PER_W = _BATCH // _NW

_mesh = plsc.VectorSubcoreMesh(core_axis_name="c", subcore_axis_name="s")


@functools.partial(
    pl.kernel,
    mesh=_mesh,
    out_type=jax.ShapeDtypeStruct((_BATCH, _VOCAB, _DIM), jnp.float32),
    scratch_types=[
        pltpu.VMEM((_VOCAB, _DIM), jnp.float32),
        pltpu.SemaphoreType.DMA,
    ],
)
def _sc_bcast(w_hbm, out_hbm, tab_v, sem):
    wid = lax.axis_index("s") * _NC + lax.axis_index("c")
    base = wid * _PER_W
    pltpu.sync_copy(w_hbm, tab_v)
    copies = [
        pltpu.make_async_copy(tab_v, out_hbm.at[base + i], sem)
        for i in range(_PER_W)
    ]
    for c in copies:
        c.start()
    for c in copies:
        c.wait()


def kernel(x, pe_weight):
    del x
    return _sc_bcast(pe_weight)


# speedup vs baseline: 5.2422x; 5.2422x over previous
import jax
import jax.numpy as jnp
from jax.experimental import pallas as pl
from jax.experimental.pallas import tpu as pltpu

_BATCH = 128
_VOCAB = 200
_DIM = 128
_EDGES = (0, 2, 4, 8, 16, 32)
_NFILL = len(_EDGES) - 1
_Q = 32
_NTAIL = _BATCH // _Q - 1


def _copy_kernel(w_ref, out_ref, buf_ref, sem):
    w = w_ref[...][None, :, :]
    for k in range(_NFILL):
        a, b = _EDGES[k], _EDGES[k + 1]
        buf_ref[pl.ds(a, b - a)] = jnp.broadcast_to(w, (b - a, _VOCAB, _DIM))
        pltpu.make_async_copy(
            buf_ref.at[pl.ds(a, b - a)],
            out_ref.at[pl.ds(a, b - a)],
            sem.at[k],
        ).start()
    for t in range(_NTAIL):
        pltpu.make_async_copy(
            buf_ref, out_ref.at[pl.ds(_Q * (t + 1), _Q)],
            sem.at[_NFILL + t]).start()
    for k in range(_NFILL):
        a, b = _EDGES[k], _EDGES[k + 1]
        pltpu.make_async_copy(
            buf_ref.at[pl.ds(a, b - a)],
            out_ref.at[pl.ds(a, b - a)],
            sem.at[k],
        ).wait()
    for t in range(_NTAIL):
        pltpu.make_async_copy(
            buf_ref, out_ref.at[pl.ds(_Q * (t + 1), _Q)],
            sem.at[_NFILL + t]).wait()


def kernel(x, pe_weight):
    del x
    return pl.pallas_call(
        _copy_kernel,
        in_specs=[pl.BlockSpec(memory_space=pltpu.MemorySpace.VMEM)],
        out_specs=pl.BlockSpec(memory_space=pltpu.MemorySpace.HBM),
        out_shape=jax.ShapeDtypeStruct((_BATCH, _VOCAB, _DIM), jnp.float32),
        scratch_shapes=[
            pltpu.VMEM((_Q, _VOCAB, _DIM), jnp.float32),
            pltpu.SemaphoreType.DMA((_NFILL + _NTAIL,)),
        ],
    )(pe_weight)
